# baseline (speedup 1.0000x reference)
import math

import jax
import jax.numpy as jnp
from jax import lax
from jax.experimental import pallas as pl
from jax.experimental.pallas import tpu as pltpu
from jax.experimental.pallas import tpu_sc as plsc

_SIZE = 100000
_N = 2048
_PAD = 0
_SMOOTH = 0.1
_EPS = _SMOOTH / (_SIZE - 2)
_CONF = 1.0 - _SMOOTH
_K = (_SIZE - 2) * _EPS * math.log(_EPS) + _CONF * math.log(_CONF)

_NC = 2
_NS = 16
_L = 16
_NW = _NC * _NS
_BPW = _N // _NW

_BLK = 2048
_GRID = (_SIZE + _BLK - 1) // _BLK


def _sc_body(xflat_hbm, tgt_hbm, c_hbm, w_hbm,
             tgt_v, idx_v, g_v, z_v, out_c, out_w, sem):
    wid = lax.axis_index("s") * _NC + lax.axis_index("c")
    base = wid * _BPW
    pltpu.sync_copy(tgt_hbm.at[pl.ds(base, _BPW)], tgt_v)
    for j in range(_BPW // _L):
        rows = base + j * _L + lax.broadcasted_iota(jnp.int32, (_L,), 0)
        tv = tgt_v[pl.ds(j * _L, _L)]
        idx_v[pl.ds(j * _L, _L)] = rows * _SIZE + tv
    pltpu.async_copy(xflat_hbm.at[idx_v], g_v, sem).wait()
    for j in range(_BPW // _L):
        rows = base + j * _L + lax.broadcasted_iota(jnp.int32, (_L,), 0)
        idx_v[pl.ds(j * _L, _L)] = rows * _SIZE
    pltpu.async_copy(xflat_hbm.at[idx_v], z_v, sem).wait()
    for j in range(_BPW // _L):
        sl = pl.ds(j * _L, _L)
        tv = tgt_v[sl]
        m = jnp.where(tv == _PAD, 0.0, 1.0)
        out_c[sl] = m * (_K + _EPS * z_v[sl] + (_EPS - _CONF) * g_v[sl])
        out_w[sl] = m * (-_EPS)
    pltpu.sync_copy(out_c, c_hbm.at[pl.ds(base, _BPW)])
    pltpu.sync_copy(out_w, w_hbm.at[pl.ds(base, _BPW)])


_sc_call = pl.kernel(
    _sc_body,
    out_type=(
        jax.ShapeDtypeStruct((_N,), jnp.float32),
        jax.ShapeDtypeStruct((_N,), jnp.float32),
    ),
    mesh=plsc.VectorSubcoreMesh(core_axis_name="c", subcore_axis_name="s"),
    scratch_types=[
        pltpu.VMEM((_BPW,), jnp.int32),
        pltpu.VMEM((_BPW,), jnp.int32),
        pltpu.VMEM((_BPW,), jnp.float32),
        pltpu.VMEM((_BPW,), jnp.float32),
        pltpu.VMEM((_BPW,), jnp.float32),
        pltpu.VMEM((_BPW,), jnp.float32),
        pltpu.SemaphoreType.DMA,
    ],
)


def _tc_body(x_ref, c_ref, w_ref, out_ref, acc_ref):
    i = pl.program_id(0)

    @pl.when(i == 0)
    def _init():
        acc_ref[...] = jnp.zeros_like(acc_ref)

    @pl.when(i < _GRID - 1)
    def _full():
        acc_ref[...] = acc_ref[...] + jnp.sum(x_ref[...], axis=1,
                                              keepdims=True)

    @pl.when(i == _GRID - 1)
    def _last():
        xblk = x_ref[...]
        cols = i * _BLK + lax.broadcasted_iota(jnp.int32, xblk.shape, 1)
        xm = jnp.where(cols < _SIZE, xblk, 0.0)
        acc = acc_ref[...] + jnp.sum(xm, axis=1, keepdims=True)
        out_ref[...] = jnp.sum(w_ref[...] * acc + c_ref[...], keepdims=True)


def kernel(x, target):
    tgt = target.astype(jnp.int32)
    c, w = _sc_call(x.reshape(-1), tgt)
    out = pl.pallas_call(
        _tc_body,
        grid=(_GRID,),
        in_specs=[
            pl.BlockSpec((_N, _BLK), lambda i: (0, i)),
            pl.BlockSpec((_N, 1), lambda i: (0, 0)),
            pl.BlockSpec((_N, 1), lambda i: (0, 0)),
        ],
        out_specs=pl.BlockSpec((1, 1), lambda i: (0, 0)),
        out_shape=jax.ShapeDtypeStruct((1, 1), jnp.float32),
        scratch_shapes=[pltpu.VMEM((_N, 1), jnp.float32)],
    )(x, c.reshape(_N, 1), w.reshape(_N, 1))
    return out.reshape(())


# speedup vs baseline: 2.0897x; 2.0897x over previous
import math

import jax
import jax.numpy as jnp
from jax import lax
from jax.experimental import pallas as pl
from jax.experimental.pallas import tpu as pltpu
from jax.experimental.pallas import tpu_sc as plsc

_SIZE = 100000
_N = 2048
_PAD = 0
_SMOOTH = 0.1
_EPS = _SMOOTH / (_SIZE - 2)
_CONF = 1.0 - _SMOOTH
_K = (_SIZE - 2) * _EPS * math.log(_EPS) + _CONF * math.log(_CONF)

_L = 16
_NC = 2

_BLK = 1024
_GRID = (_SIZE + _BLK - 1) // _BLK


def _tc_body(x_ref, t_ref, s_ref, g_ref):
    i = pl.program_id(0)
    xblk = x_ref[...]
    cols = i * _BLK + lax.broadcasted_iota(jnp.int32, xblk.shape, 1)
    t = t_ref[...]

    @pl.when(i == 0)
    def _init():
        s_ref[...] = -xblk[:, 0:1]
        g_ref[...] = jnp.zeros_like(g_ref)

    @pl.when(i < _GRID - 1)
    def _interior():
        s_ref[...] += jnp.sum(xblk, axis=1, keepdims=True)
        g_ref[...] += jnp.sum(jnp.where(cols == t, xblk, 0.0), axis=1,
                              keepdims=True)

    @pl.when(i == _GRID - 1)
    def _last():
        xm = jnp.where(cols < _SIZE, xblk, 0.0)
        s_ref[...] += jnp.sum(xm, axis=1, keepdims=True)
        g_ref[...] += jnp.sum(jnp.where(cols == t, xm, 0.0), axis=1,
                              keepdims=True)


def _sc_body(s_hbm, g_hbm, t_hbm, out_hbm, s_v, g_v, t_v, out_v):
    wid = lax.axis_index("s") * _NC + lax.axis_index("c")

    @pl.when(wid == 0)
    def _combine():
        pltpu.sync_copy(s_hbm, s_v)
        pltpu.sync_copy(g_hbm, g_v)
        pltpu.sync_copy(t_hbm, t_v)

        def body(k, acc):
            sl = pl.ds(k * _L, _L)
            m = jnp.where(t_v[sl] == _PAD, 0.0, 1.0)
            g = g_v[sl]
            return acc + m * (_K - _EPS * (s_v[sl] - g) - _CONF * g)

        out_v[...] = lax.fori_loop(0, _N // _L, body,
                                   jnp.zeros((_L,), jnp.float32))
        pltpu.sync_copy(out_v, out_hbm)


_sc_call = pl.kernel(
    _sc_body,
    out_type=jax.ShapeDtypeStruct((_L,), jnp.float32),
    mesh=plsc.VectorSubcoreMesh(core_axis_name="c", subcore_axis_name="s"),
    scratch_types=[
        pltpu.VMEM((_N,), jnp.float32),
        pltpu.VMEM((_N,), jnp.float32),
        pltpu.VMEM((_N,), jnp.int32),
        pltpu.VMEM((_L,), jnp.float32),
    ],
)


def kernel(x, target):
    t2d = target.astype(jnp.int32).reshape(_N, 1)
    s, g = pl.pallas_call(
        _tc_body,
        grid=(_GRID,),
        in_specs=[
            pl.BlockSpec((_N, _BLK), lambda i: (0, i)),
            pl.BlockSpec((_N, 1), lambda i: (0, 0)),
        ],
        out_specs=[
            pl.BlockSpec((_N, 1), lambda i: (0, 0)),
            pl.BlockSpec((_N, 1), lambda i: (0, 0)),
        ],
        out_shape=[
            jax.ShapeDtypeStruct((_N, 1), jnp.float32),
            jax.ShapeDtypeStruct((_N, 1), jnp.float32),
        ],
    )(x, t2d)
    out = _sc_call(s.reshape(-1), g.reshape(-1), target.astype(jnp.int32))
    return jnp.sum(out)


# speedup vs baseline: 2.1724x; 1.0396x over previous
import math

import jax
import jax.numpy as jnp
from jax import lax
from jax.experimental import pallas as pl
from jax.experimental.pallas import tpu as pltpu
from jax.experimental.pallas import tpu_sc as plsc

_SIZE = 100000
_N = 2048
_PAD = 0
_SMOOTH = 0.1
_EPS = _SMOOTH / (_SIZE - 2)
_CONF = 1.0 - _SMOOTH
_K = (_SIZE - 2) * _EPS * math.log(_EPS) + _CONF * math.log(_CONF)

_L = 16
_NC = 2

_BLK = 2048
_CW = 512
_GRID = (_SIZE + _BLK - 1) // _BLK


def _tc_body(x_ref, t_ref, p_ref, acc_ref):
    i = pl.program_id(0)
    t = t_ref[...]

    @pl.when(i == 0)
    def _init():
        acc_ref[...] = jnp.zeros_like(acc_ref)
        acc_ref[:, 0:1] = _EPS * x_ref[:, 0:1]

    @pl.when(i < _GRID - 1)
    def _interior():
        for k in range(_BLK // _CW):
            xc = x_ref[:, k * _CW:(k + 1) * _CW]
            cols = (i * _BLK + k * _CW) + lax.broadcasted_iota(
                jnp.int32, xc.shape, 1)
            coef = jnp.where(cols == t, -_CONF, -_EPS)
            acc_ref[...] += coef * xc

    @pl.when(i == _GRID - 1)
    def _last():
        for k in range(_BLK // _CW):
            xc = x_ref[:, k * _CW:(k + 1) * _CW]
            cols = (i * _BLK + k * _CW) + lax.broadcasted_iota(
                jnp.int32, xc.shape, 1)
            xm = jnp.where(cols < _SIZE, xc, 0.0)
            coef = jnp.where(cols == t, -_CONF, -_EPS)
            acc_ref[...] += coef * xm
        p_ref[...] = jnp.sum(acc_ref[...], axis=1, keepdims=True)


def _sc_body(p_hbm, t_hbm, out_hbm, p_v, t_v, out_v):
    wid = lax.axis_index("s") * _NC + lax.axis_index("c")

    @pl.when(wid == 0)
    def _combine():
        pltpu.sync_copy(p_hbm, p_v)
        pltpu.sync_copy(t_hbm, t_v)

        def body(k, acc):
            sl = pl.ds(k * _L, _L)
            m = jnp.where(t_v[sl] == _PAD, 0.0, 1.0)
            return acc + m * (_K + p_v[sl])

        out_v[...] = lax.fori_loop(0, _N // _L, body,
                                   jnp.zeros((_L,), jnp.float32))
        pltpu.sync_copy(out_v, out_hbm)


def _make_sc_call():
    return pl.kernel(
        _sc_body,
        out_type=jax.ShapeDtypeStruct((_L,), jnp.float32),
        mesh=plsc.VectorSubcoreMesh(core_axis_name="c", subcore_axis_name="s"),
        scratch_types=[
            pltpu.VMEM((_N,), jnp.float32),
            pltpu.VMEM((_N,), jnp.int32),
            pltpu.VMEM((_L,), jnp.float32),
        ],
    )


def kernel(x, target):
    t2d = target.astype(jnp.int32).reshape(_N, 1)
    p = pl.pallas_call(
        _tc_body,
        grid=(_GRID,),
        in_specs=[
            pl.BlockSpec((_N, _BLK), lambda i: (0, i)),
            pl.BlockSpec((_N, 1), lambda i: (0, 0)),
        ],
        out_specs=pl.BlockSpec((_N, 1), lambda i: (0, 0)),
        out_shape=jax.ShapeDtypeStruct((_N, 1), jnp.float32),
        scratch_shapes=[pltpu.VMEM((_N, _CW), jnp.float32)],
    )(x, t2d)
    out = _make_sc_call()(p.reshape(-1), target.astype(jnp.int32))
    return jnp.sum(out)


# speedup vs baseline: 2.1764x; 1.0018x over previous
import math

import jax
import jax.numpy as jnp
from jax import lax
from jax.experimental import pallas as pl
from jax.experimental.pallas import tpu as pltpu
from jax.experimental.pallas import tpu_sc as plsc

_SIZE = 100000
_N = 2048
_PAD = 0
_SMOOTH = 0.1
_EPS = _SMOOTH / (_SIZE - 2)
_CONF = 1.0 - _SMOOTH
_K = (_SIZE - 2) * _EPS * math.log(_EPS) + _CONF * math.log(_CONF)

_L = 16
_NC = 2

_BLK = 1024
_CW = 512
_GRID = 49
_HALF = _GRID * _BLK


def _accum(acc_ref, x_ref, t, base, masked):
    for k in range(_BLK // _CW):
        xc = x_ref[:, k * _CW:(k + 1) * _CW]
        cols = (base + k * _CW) + lax.broadcasted_iota(
            jnp.int32, xc.shape, 1)
        if masked:
            xc = jnp.where(cols < _SIZE, xc, 0.0)
        coef = jnp.where(cols == t, -_CONF, -_EPS)
        acc_ref[...] += coef * xc


def _tc_body(xa_ref, xb_ref, t_ref, p_ref, acc_ref):
    i = pl.program_id(0)
    t = t_ref[...]

    @pl.when(i == 0)
    def _init():
        acc_ref[...] = jnp.zeros_like(acc_ref)
        acc_ref[:, 0:1] = _EPS * xa_ref[:, 0:1]

    @pl.when(i < _GRID - 1)
    def _interior():
        _accum(acc_ref, xa_ref, t, i * _BLK, masked=False)
        _accum(acc_ref, xb_ref, t, _HALF + i * _BLK, masked=False)

    @pl.when(i == _GRID - 1)
    def _last():
        _accum(acc_ref, xa_ref, t, i * _BLK, masked=False)
        _accum(acc_ref, xb_ref, t, _HALF + i * _BLK, masked=True)
        p_ref[...] = jnp.sum(acc_ref[...], axis=1, keepdims=True)


def _sc_body(p_hbm, t_hbm, out_hbm, p_v, t_v, out_v):
    wid = lax.axis_index("s") * _NC + lax.axis_index("c")

    @pl.when(wid == 0)
    def _combine():
        pltpu.sync_copy(p_hbm, p_v)
        pltpu.sync_copy(t_hbm, t_v)

        def body(k, acc):
            sl = pl.ds(k * _L, _L)
            m = jnp.where(t_v[sl] == _PAD, 0.0, 1.0)
            return acc + m * (_K + p_v[sl])

        out_v[...] = lax.fori_loop(0, _N // _L, body,
                                   jnp.zeros((_L,), jnp.float32))
        pltpu.sync_copy(out_v, out_hbm)


def _make_sc_call():
    return pl.kernel(
        _sc_body,
        out_type=jax.ShapeDtypeStruct((_L,), jnp.float32),
        mesh=plsc.VectorSubcoreMesh(core_axis_name="c", subcore_axis_name="s"),
        scratch_types=[
            pltpu.VMEM((_N,), jnp.float32),
            pltpu.VMEM((_N,), jnp.int32),
            pltpu.VMEM((_L,), jnp.float32),
        ],
    )


def kernel(x, target):
    t2d = target.astype(jnp.int32).reshape(_N, 1)
    p = pl.pallas_call(
        _tc_body,
        grid=(_GRID,),
        in_specs=[
            pl.BlockSpec((_N, _BLK), lambda i: (0, i)),
            pl.BlockSpec((_N, _BLK), lambda i: (0, i + _GRID)),
            pl.BlockSpec((_N, 1), lambda i: (0, 0)),
        ],
        out_specs=pl.BlockSpec((_N, 1), lambda i: (0, 0)),
        out_shape=jax.ShapeDtypeStruct((_N, 1), jnp.float32),
        scratch_shapes=[pltpu.VMEM((_N, _CW), jnp.float32)],
    )(x, x, t2d)
    out = _make_sc_call()(p.reshape(-1), target.astype(jnp.int32))
    return jnp.sum(out)
